# baseline (speedup 1.0000x reference)
import jax
import jax.numpy as jnp
from jax.experimental import pallas as pl

B, T, D, NCLS = 32, 512, 512, 20
K_EASY, K_HARD = 20, 10
M_BIG, M_SMALL = 5, 3
A4, A5, A6, A7 = 1.0, 0.5, 0.5, 0.1

NB = 8

_DN = (((1,), (1,)), ((), ()))
_MM = (((1,), (0,)), ((), ()))


def _shift(x, o):
    if o == 0:
        return x
    z = jnp.zeros((1, abs(o)), x.dtype)
    if o > 0:
        return jnp.concatenate([x[:, o:], z], axis=1)
    return jnp.concatenate([z, x[:, :T + o]], axis=1)


def _erode(x, w):
    c = w // 2
    out = x
    for o in range(-c, c + 1):
        if o != 0:
            out = jnp.minimum(out, _shift(x, o))
    return out


def _dilate(x, w):
    c = w // 2
    out = x
    for o in range(-c, c + 1):
        if o != 0:
            out = jnp.maximum(out, _shift(x, o))
    return out


def _gmat(base_col, base_row, ltb):
    one = jnp.ones((), jnp.bfloat16)
    zero = jnp.zeros((), jnp.bfloat16)
    return jnp.where(base_col > base_row, one,
                     jnp.where(base_col == base_row, ltb, zero))


def _rank_from(z, u, w):
    nz = jnp.sum(z)
    return z * u + (1.0 - z) * (nz + w)


def _sel_rows(rank, k):
    r = rank.astype(jnp.int32)
    kk = jax.lax.broadcasted_iota(jnp.int32, (k, T), 0)
    return (r == kk).astype(jnp.float32)


def _one_batch(a, drop, mu, var, cf, ltb):
    bf = jnp.bfloat16
    a_col = jnp.transpose(a, (1, 0))

    amax = jnp.max(a)
    rev = amax - a
    rev_col = amax - a_col
    ahat = a * 2.5
    ahat_col = a_col * 2.5
    bhat = rev * 2.5
    bhat_col = rev_col * 2.5

    g_a_bf = _gmat(a_col, a, ltb)
    g_ea = _gmat(ahat_col, ahat, ltb)
    g_eb = _gmat(bhat_col, bhat, ltb)

    ones_bf = jnp.ones((1, T), bf)
    rank_a = jax.lax.dot_general(ones_bf, g_a_bf, _MM,
                                 preferred_element_type=jnp.float32)
    r_a = rank_a.astype(jnp.int32)
    v_lo = jnp.sum(jnp.where(r_a == (T // 2), a, 0.0))
    v_hi = jnp.sum(jnp.where(r_a == (T // 2 - 1), a, 0.0))
    med = 0.5 * (v_lo + v_hi)

    abin = jnp.where(a > med, 1.0, 0.0)
    idx_inner = _erode(abin, M_SMALL) - _erode(abin, M_BIG)
    idx_outer = _dilate(abin, M_BIG) - _dilate(abin, M_SMALL)

    z_ea = jnp.where((drop > 0.0) & (a > 0.0), 1.0, 0.0)
    z_eb = jnp.where((drop > 0.0) & (rev > 0.0), 1.0, 0.0)
    z_ha = jnp.where((idx_inner > 0.0) & (a > 0.0), 1.0, 0.0)
    z_hb = jnp.where((idx_outer > 0.0) & (a > 0.0), 1.0, 0.0)

    u_ea = jax.lax.dot_general(z_ea.astype(bf), g_ea, _MM,
                               preferred_element_type=jnp.float32)
    u_eb = jax.lax.dot_general(z_eb.astype(bf), g_eb, _MM,
                               preferred_element_type=jnp.float32)
    zh = jnp.concatenate([z_ha, z_hb], axis=0).astype(bf)
    uh = jax.lax.dot_general(zh, g_a_bf, _MM,
                             preferred_element_type=jnp.float32)
    zn = jnp.concatenate([1.0 - z_ea, 1.0 - z_eb, 1.0 - z_ha, 1.0 - z_hb],
                         axis=0).astype(bf)
    wz = jax.lax.dot_general(zn, ltb, _MM,
                             preferred_element_type=jnp.float32)

    rank_ea = _rank_from(z_ea, u_ea, wz[0:1])
    rank_eb = _rank_from(z_eb, u_eb, wz[1:2])
    rank_ha = _rank_from(z_ha, uh[0:1], wz[2:3])
    rank_hb = _rank_from(z_hb, uh[1:2], wz[3:4])

    S = jnp.concatenate([
        _sel_rows(rank_ea, K_EASY),
        _sel_rows(rank_eb, K_EASY),
        _sel_rows(rank_ha, K_HARD),
        _sel_rows(rank_hb, K_HARD),
    ], axis=0)
    g_mu = jnp.dot(S, mu)
    g_var = jnp.dot(S, var)

    q_mu, p_mu = g_mu[:2 * K_EASY], g_mu[2 * K_EASY:]
    q_var, p_var = g_var[:2 * K_EASY], g_var[2 * K_EASY:]
    cq = q_var + 1e-5
    cp = p_var + 1e-5
    rq = 1.0 / cq
    slog_q = jnp.sum(jnp.log(cq), axis=-1)
    slog_p = jnp.sum(jnp.log(cp), axis=-1)
    q2r = jnp.sum(q_mu * q_mu * rq, axis=-1)
    m1 = jax.lax.dot_general(p_mu, q_mu * rq, _DN)
    m2 = jax.lax.dot_general(p_mu * p_mu, rq, _DN)
    m3 = jax.lax.dot_general(cp, rq, _DN)
    t1 = q2r[None, :] - 2.0 * m1 + m2
    t2 = slog_q[None, :] - slog_p[:, None]
    dist = 0.5 * (t1 + t2 + m3) - 0.5 * D
    val = 1.0 / (dist + 1.0)
    pos_a = jnp.mean(val[:K_HARD, :K_EASY])
    neg_a = jnp.mean(val[:K_HARD, K_EASY:])
    pos_b = jnp.mean(val[K_HARD:, K_EASY:])
    neg_b = jnp.mean(val[K_HARD:, :K_EASY])
    la = jnp.log(pos_a) + jnp.log(1.0 - neg_a)
    lb = jnp.log(pos_b) + jnp.log(1.0 - neg_b)

    ones_w = jnp.ones((2 * (D // 4), 128), bf)

    def _rowsum(x):
        p = (x[:, 0:128] + x[:, 128:256]) + (x[:, 256:384] + x[:, 384:512])
        hi = p.astype(bf)
        lo = (p - hi.astype(jnp.float32)).astype(bf)
        hl = jnp.concatenate([hi, lo], axis=1)
        return jax.lax.dot_general(hl, ones_w, _MM,
                                   preferred_element_type=jnp.float32)

    s1 = _rowsum(mu * cf)
    n1 = jnp.maximum(jnp.sqrt(_rowsum(mu * mu)), 1e-12)
    n2 = jnp.maximum(jnp.sqrt(_rowsum(cf * cf)), 1e-12)
    sim = (s1 / (n1 * n2) + 1.0) * 0.5
    lane0 = jax.lax.broadcasted_iota(jnp.int32, (T, 128), 1)
    sim_sum = jnp.sum(jnp.where(lane0 == 0, sim, 0.0))
    return sim_sum, la, lb


def _loss_kernel(ltb_ref, act_ref, drop_ref, mu_ref, var_ref,
                 clip_ref, tf_ref, out_ref):
    b = pl.program_id(0)

    @pl.when(b == 0)
    def _init():
        tf = tf_ref[...]
        tn = jnp.maximum(
            jnp.sqrt(jnp.sum(tf * tf, axis=-1, keepdims=True)), 1e-12)
        e = tf / tn
        g = jax.lax.dot_general(e, e, _DN,
                                precision=jax.lax.Precision.HIGHEST)
        ii = jax.lax.broadcasted_iota(jnp.int32, (NCLS, NCLS), 0)
        jj = jax.lax.broadcasted_iota(jnp.int32, (NCLS, NCLS), 1)
        g = g - jnp.where(ii == jj, 1.0, 0.0)
        ortho = jnp.sqrt(jnp.sum(g * g))
        lane0 = jax.lax.broadcasted_iota(jnp.int32, (1, 8), 1)
        out_ref[...] = ortho * (lane0 == 3)

    ltb = ltb_ref[...]
    sim_sum = 0.0
    la = 0.0
    lb = 0.0
    for i in range(NB):
        s, x, y = _one_batch(act_ref[i], drop_ref[i], mu_ref[i],
                             var_ref[i], clip_ref[i], ltb)
        sim_sum += s
        la += x
        lb += y

    lane = jax.lax.broadcasted_iota(jnp.int32, (1, 8), 1)
    contrib = (sim_sum * (lane == 0) + la * (lane == 1) + lb * (lane == 2))
    out_ref[...] = out_ref[...] + contrib


@jax.jit
def kernel(attn, mu_v, var_v, text_feat, mu_clip, labels, drop_mask):
    del labels
    act = attn.reshape(B, 1, T)
    drop3 = drop_mask.reshape(B, 1, T)
    ii = jax.lax.broadcasted_iota(jnp.int32, (T, T), 0)
    jj = jax.lax.broadcasted_iota(jnp.int32, (T, T), 1)
    ltf = (ii < jj).astype(jnp.float32)
    ltb = ltf.astype(jnp.bfloat16)
    acc = pl.pallas_call(
        _loss_kernel,
        grid=(B // NB,),
        in_specs=[
            pl.BlockSpec((T, T), lambda b: (0, 0)),
            pl.BlockSpec((NB, 1, T), lambda b: (b, 0, 0)),
            pl.BlockSpec((NB, 1, T), lambda b: (b, 0, 0)),
            pl.BlockSpec((NB, T, D), lambda b: (b, 0, 0)),
            pl.BlockSpec((NB, T, D), lambda b: (b, 0, 0)),
            pl.BlockSpec((NB, T, D), lambda b: (b, 0, 0)),
            pl.BlockSpec((NCLS, D), lambda b: (0, 0)),
        ],
        out_specs=pl.BlockSpec((1, 8), lambda b: (0, 0)),
        out_shape=jax.ShapeDtypeStruct((1, 8), jnp.float32),
    )(ltb, act, drop3, mu_v, var_v, mu_clip, text_feat)

    d_loss = A4 * -jnp.log(acc[0, 0] / (B * T))
    a_loss = A5 * (-acc[0, 1] / B)
    b_loss = A6 * (-acc[0, 2] / B)
    o_loss = A7 * acc[0, 3]
    total = d_loss + a_loss + b_loss + o_loss
    return total, d_loss, a_loss, b_loss, o_loss


# speedup vs baseline: 1.1007x; 1.1007x over previous
import jax
import jax.numpy as jnp
from jax.experimental import pallas as pl

B, T, D, NCLS = 32, 512, 512, 20
K_EASY, K_HARD = 20, 10
M_BIG, M_SMALL = 5, 3
A4, A5, A6, A7 = 1.0, 0.5, 0.5, 0.1

NB = 4

_DN = (((1,), (1,)), ((), ()))
_MM = (((1,), (0,)), ((), ()))


def _shift(x, o):
    if o == 0:
        return x
    z = jnp.zeros((x.shape[0], abs(o)), x.dtype)
    if o > 0:
        return jnp.concatenate([x[:, o:], z], axis=1)
    return jnp.concatenate([z, x[:, :T + o]], axis=1)


def _erode(x, w):
    c = w // 2
    out = x
    for o in range(-c, c + 1):
        if o != 0:
            out = jnp.minimum(out, _shift(x, o))
    return out


def _dilate(x, w):
    c = w // 2
    out = x
    for o in range(-c, c + 1):
        if o != 0:
            out = jnp.maximum(out, _shift(x, o))
    return out


def _gmat(base_col, base_row, ltb):
    one = jnp.ones((), jnp.bfloat16)
    zero = jnp.zeros((), jnp.bfloat16)
    return jnp.where(base_col > base_row, one,
                     jnp.where(base_col == base_row, ltb, zero))


def _rank_from(z, u, w):
    nz = jnp.sum(z, axis=1, keepdims=True)
    return z * u + (1.0 - z) * (nz + w)


def _sel_rows(rank, k):
    r = rank.astype(jnp.int32)
    kk = jax.lax.broadcasted_iota(jnp.int32, (k, T), 0)
    return (r == kk).astype(jnp.float32)


def _step(aa, drop, mu3, var3, cf3, ltb):
    bf = jnp.bfloat16
    f32 = jnp.float32
    a_colall = jnp.transpose(aa, (1, 0))

    amax = jnp.max(aa, axis=1, keepdims=True)
    rev = amax - aa
    rev_colall = jnp.transpose(amax, (1, 0)) - a_colall
    ahat = aa * 2.5
    ahat_colall = a_colall * 2.5
    bhat = rev * 2.5
    bhat_colall = rev_colall * 2.5

    g_a = [_gmat(a_colall[:, i:i + 1], aa[i:i + 1], ltb) for i in range(NB)]
    g_ea = [_gmat(ahat_colall[:, i:i + 1], ahat[i:i + 1], ltb)
            for i in range(NB)]
    g_eb = [_gmat(bhat_colall[:, i:i + 1], bhat[i:i + 1], ltb)
            for i in range(NB)]

    ones_bf = jnp.ones((1, T), bf)
    rank_a = jnp.concatenate([
        jax.lax.dot_general(ones_bf, g_a[i], _MM, preferred_element_type=f32)
        for i in range(NB)], axis=0)
    r_a = rank_a.astype(jnp.int32)
    v_lo = jnp.sum(jnp.where(r_a == (T // 2), aa, 0.0), axis=1, keepdims=True)
    v_hi = jnp.sum(jnp.where(r_a == (T // 2 - 1), aa, 0.0), axis=1,
                   keepdims=True)
    med = 0.5 * (v_lo + v_hi)

    abin = jnp.where(aa > med, 1.0, 0.0)
    idx_inner = _erode(abin, M_SMALL) - _erode(abin, M_BIG)
    idx_outer = _dilate(abin, M_BIG) - _dilate(abin, M_SMALL)

    z_ea = jnp.where((drop > 0.0) & (aa > 0.0), 1.0, 0.0)
    z_eb = jnp.where((drop > 0.0) & (rev > 0.0), 1.0, 0.0)
    z_ha = jnp.where((idx_inner > 0.0) & (aa > 0.0), 1.0, 0.0)
    z_hb = jnp.where((idx_outer > 0.0) & (aa > 0.0), 1.0, 0.0)

    u_ea = jnp.concatenate([
        jax.lax.dot_general(z_ea[i:i + 1].astype(bf), g_ea[i], _MM,
                            preferred_element_type=f32)
        for i in range(NB)], axis=0)
    u_eb = jnp.concatenate([
        jax.lax.dot_general(z_eb[i:i + 1].astype(bf), g_eb[i], _MM,
                            preferred_element_type=f32)
        for i in range(NB)], axis=0)
    uh = [jax.lax.dot_general(
        jnp.concatenate([z_ha[i:i + 1], z_hb[i:i + 1]], axis=0).astype(bf),
        g_a[i], _MM, preferred_element_type=f32) for i in range(NB)]
    u_ha = jnp.concatenate([u[0:1] for u in uh], axis=0)
    u_hb = jnp.concatenate([u[1:2] for u in uh], axis=0)
    zn = jnp.concatenate([1.0 - z_ea, 1.0 - z_eb, 1.0 - z_ha, 1.0 - z_hb],
                         axis=0).astype(bf)
    wz = jax.lax.dot_general(zn, ltb, _MM,
                             preferred_element_type=f32)

    rank_ea = _rank_from(z_ea, u_ea, wz[0 * NB:1 * NB])
    rank_eb = _rank_from(z_eb, u_eb, wz[1 * NB:2 * NB])
    rank_ha = _rank_from(z_ha, u_ha, wz[2 * NB:3 * NB])
    rank_hb = _rank_from(z_hb, u_hb, wz[3 * NB:4 * NB])

    g_mu = []
    g_var = []
    for i in range(NB):
        S = jnp.concatenate([
            _sel_rows(rank_ea[i:i + 1], K_EASY),
            _sel_rows(rank_eb[i:i + 1], K_EASY),
            _sel_rows(rank_ha[i:i + 1], K_HARD),
            _sel_rows(rank_hb[i:i + 1], K_HARD),
        ], axis=0)
        g_mu.append(jnp.dot(S, mu3[i]))
        g_var.append(jnp.dot(S, var3[i]))

    q_mu = jnp.concatenate([g[:2 * K_EASY] for g in g_mu], axis=0)
    p_mu = jnp.concatenate([g[2 * K_EASY:] for g in g_mu], axis=0)
    q_var = jnp.concatenate([g[:2 * K_EASY] for g in g_var], axis=0)
    p_var = jnp.concatenate([g[2 * K_EASY:] for g in g_var], axis=0)
    nq, np_ = 2 * K_EASY, 2 * K_HARD
    cq = q_var + 1e-5
    cp = p_var + 1e-5
    rq = 1.0 / cq
    slog_q = jnp.sum(jnp.log(cq), axis=-1)
    slog_p = jnp.sum(jnp.log(cp), axis=-1)
    q2r = jnp.sum(q_mu * q_mu * rq, axis=-1)
    m1 = jax.lax.dot_general(p_mu, q_mu * rq, _DN)
    m2 = jax.lax.dot_general(p_mu * p_mu, rq, _DN)
    m3 = jax.lax.dot_general(cp, rq, _DN)
    la = 0.0
    lb = 0.0
    for i in range(NB):
        rs, cs = i * np_, i * nq
        q2r_i = q2r[cs:cs + nq]
        t2 = slog_q[None, cs:cs + nq] - slog_p[rs:rs + np_, None]
        t1 = (q2r_i[None, :] - 2.0 * m1[rs:rs + np_, cs:cs + nq]
              + m2[rs:rs + np_, cs:cs + nq])
        dist = 0.5 * (t1 + t2 + m3[rs:rs + np_, cs:cs + nq]) - 0.5 * D
        val = 1.0 / (dist + 1.0)
        pos_a = jnp.mean(val[:K_HARD, :K_EASY])
        neg_a = jnp.mean(val[:K_HARD, K_EASY:])
        pos_b = jnp.mean(val[K_HARD:, K_EASY:])
        neg_b = jnp.mean(val[K_HARD:, :K_EASY])
        la += jnp.log(pos_a) + jnp.log(1.0 - neg_a)
        lb += jnp.log(pos_b) + jnp.log(1.0 - neg_b)

    ones_w = jnp.ones((2 * (D // 4), 128), bf)

    def _rowsum(x):
        p = (x[:, 0:128] + x[:, 128:256]) + (x[:, 256:384] + x[:, 384:512])
        hi = p.astype(bf)
        lo = (p - hi.astype(f32)).astype(bf)
        hl = jnp.concatenate([hi, lo], axis=1)
        return jax.lax.dot_general(hl, ones_w, _MM,
                                   preferred_element_type=f32)

    mu = mu3.reshape(NB * T, D)
    cf = cf3.reshape(NB * T, D)
    s1 = _rowsum(mu * cf)
    n1 = jnp.maximum(jnp.sqrt(_rowsum(mu * mu)), 1e-12)
    n2 = jnp.maximum(jnp.sqrt(_rowsum(cf * cf)), 1e-12)
    sim = (s1 / (n1 * n2) + 1.0) * 0.5
    lane0 = jax.lax.broadcasted_iota(jnp.int32, (NB * T, 128), 1)
    sim_sum = jnp.sum(jnp.where(lane0 == 0, sim, 0.0))
    return sim_sum, la, lb


def _loss_kernel(ltb_ref, act_ref, drop_ref, mu_ref, var_ref,
                 clip_ref, tf_ref, out_ref):
    b = pl.program_id(0)

    @pl.when(b == 0)
    def _init():
        tf = tf_ref[...]
        tn = jnp.maximum(
            jnp.sqrt(jnp.sum(tf * tf, axis=-1, keepdims=True)), 1e-12)
        e = tf / tn
        g = jax.lax.dot_general(e, e, _DN,
                                precision=jax.lax.Precision.HIGHEST)
        ii = jax.lax.broadcasted_iota(jnp.int32, (NCLS, NCLS), 0)
        jj = jax.lax.broadcasted_iota(jnp.int32, (NCLS, NCLS), 1)
        g = g - jnp.where(ii == jj, 1.0, 0.0)
        ortho = jnp.sqrt(jnp.sum(g * g))
        lane0 = jax.lax.broadcasted_iota(jnp.int32, (1, 8), 1)
        out_ref[...] = ortho * (lane0 == 3)

    sim_sum, la, lb = _step(act_ref[:, 0, :], drop_ref[:, 0, :], mu_ref[...],
                            var_ref[...], clip_ref[...], ltb_ref[...])

    lane = jax.lax.broadcasted_iota(jnp.int32, (1, 8), 1)
    contrib = (sim_sum * (lane == 0) + la * (lane == 1) + lb * (lane == 2))
    out_ref[...] = out_ref[...] + contrib


@jax.jit
def kernel(attn, mu_v, var_v, text_feat, mu_clip, labels, drop_mask):
    del labels
    act = attn.reshape(B, 1, T)
    drop3 = drop_mask.reshape(B, 1, T)
    ii = jax.lax.broadcasted_iota(jnp.int32, (T, T), 0)
    jj = jax.lax.broadcasted_iota(jnp.int32, (T, T), 1)
    ltf = (ii < jj).astype(jnp.float32)
    ltb = ltf.astype(jnp.bfloat16)
    acc = pl.pallas_call(
        _loss_kernel,
        grid=(B // NB,),
        in_specs=[
            pl.BlockSpec((T, T), lambda b: (0, 0)),
            pl.BlockSpec((NB, 1, T), lambda b: (b, 0, 0)),
            pl.BlockSpec((NB, 1, T), lambda b: (b, 0, 0)),
            pl.BlockSpec((NB, T, D), lambda b: (b, 0, 0)),
            pl.BlockSpec((NB, T, D), lambda b: (b, 0, 0)),
            pl.BlockSpec((NB, T, D), lambda b: (b, 0, 0)),
            pl.BlockSpec((NCLS, D), lambda b: (0, 0)),
        ],
        out_specs=pl.BlockSpec((1, 8), lambda b: (0, 0)),
        out_shape=jax.ShapeDtypeStruct((1, 8), jnp.float32),
    )(ltb, act, drop3, mu_v, var_v, mu_clip, text_feat)

    d_loss = A4 * -jnp.log(acc[0, 0] / (B * T))
    a_loss = A5 * (-acc[0, 1] / B)
    b_loss = A6 * (-acc[0, 2] / B)
    o_loss = A7 * acc[0, 3]
    total = d_loss + a_loss + b_loss + o_loss
    return total, d_loss, a_loss, b_loss, o_loss


# speedup vs baseline: 1.1058x; 1.0047x over previous
import jax
import jax.numpy as jnp
from jax.experimental import pallas as pl

B, T, D, NCLS = 32, 512, 512, 20
K_EASY, K_HARD = 20, 10
M_BIG, M_SMALL = 5, 3
A4, A5, A6, A7 = 1.0, 0.5, 0.5, 0.1

NB = 4

_DN = (((1,), (1,)), ((), ()))
_MM = (((1,), (0,)), ((), ()))


def _shift(x, o):
    if o == 0:
        return x
    z = jnp.zeros((x.shape[0], abs(o)), x.dtype)
    if o > 0:
        return jnp.concatenate([x[:, o:], z], axis=1)
    return jnp.concatenate([z, x[:, :T + o]], axis=1)


def _erode(x, w):
    c = w // 2
    out = x
    for o in range(-c, c + 1):
        if o != 0:
            out = jnp.minimum(out, _shift(x, o))
    return out


def _dilate(x, w):
    c = w // 2
    out = x
    for o in range(-c, c + 1):
        if o != 0:
            out = jnp.maximum(out, _shift(x, o))
    return out


def _gmat(base_col, base_row, ltb):
    one = jnp.ones((), jnp.bfloat16)
    zero = jnp.zeros((), jnp.bfloat16)
    return jnp.where(base_col > base_row, one,
                     jnp.where(base_col == base_row, ltb, zero))


def _rank_from(z, u, w):
    nz = jnp.sum(z, axis=1, keepdims=True)
    return z * u + (1.0 - z) * (nz + w)


def _sel_rows(rank, k):
    r = rank.astype(jnp.int32)
    kk = jax.lax.broadcasted_iota(jnp.int32, (k, T), 0)
    return (r == kk).astype(jnp.float32)


def _step(aa, drop, mu3, var3, cf3, ltb):
    bf = jnp.bfloat16
    f32 = jnp.float32
    a_colall = jnp.transpose(aa, (1, 0))

    amax = jnp.max(aa, axis=1, keepdims=True)
    rev = amax - aa
    rev_colall = jnp.transpose(amax, (1, 0)) - a_colall
    ahat = aa * 2.5
    ahat_colall = a_colall * 2.5
    bhat = rev * 2.5
    bhat_colall = rev_colall * 2.5

    g_a = [_gmat(a_colall[:, i:i + 1], aa[i:i + 1], ltb) for i in range(NB)]
    g_ea = [_gmat(ahat_colall[:, i:i + 1], ahat[i:i + 1], ltb)
            for i in range(NB)]
    g_eb = [_gmat(bhat_colall[:, i:i + 1], bhat[i:i + 1], ltb)
            for i in range(NB)]

    ones_bf = jnp.ones((1, T), bf)
    rank_a = jnp.concatenate([
        jax.lax.dot_general(ones_bf, g_a[i], _MM, preferred_element_type=f32)
        for i in range(NB)], axis=0)
    r_a = rank_a.astype(jnp.int32)
    v_lo = jnp.sum(jnp.where(r_a == (T // 2), aa, 0.0), axis=1, keepdims=True)
    v_hi = jnp.sum(jnp.where(r_a == (T // 2 - 1), aa, 0.0), axis=1,
                   keepdims=True)
    med = 0.5 * (v_lo + v_hi)

    abin = jnp.where(aa > med, 1.0, 0.0)
    idx_inner = _erode(abin, M_SMALL) - _erode(abin, M_BIG)
    idx_outer = _dilate(abin, M_BIG) - _dilate(abin, M_SMALL)

    z_ea = jnp.where((drop > 0.0) & (aa > 0.0), 1.0, 0.0)
    z_eb = jnp.where((drop > 0.0) & (rev > 0.0), 1.0, 0.0)
    z_ha = jnp.where((idx_inner > 0.0) & (aa > 0.0), 1.0, 0.0)
    z_hb = jnp.where((idx_outer > 0.0) & (aa > 0.0), 1.0, 0.0)

    u_ea = jnp.concatenate([
        jax.lax.dot_general(z_ea[i:i + 1].astype(bf), g_ea[i], _MM,
                            preferred_element_type=f32)
        for i in range(NB)], axis=0)
    u_eb = jnp.concatenate([
        jax.lax.dot_general(z_eb[i:i + 1].astype(bf), g_eb[i], _MM,
                            preferred_element_type=f32)
        for i in range(NB)], axis=0)
    uh = [jax.lax.dot_general(
        jnp.concatenate([z_ha[i:i + 1], z_hb[i:i + 1]], axis=0).astype(bf),
        g_a[i], _MM, preferred_element_type=f32) for i in range(NB)]
    u_ha = jnp.concatenate([u[0:1] for u in uh], axis=0)
    u_hb = jnp.concatenate([u[1:2] for u in uh], axis=0)
    zn = jnp.concatenate([1.0 - z_ea, 1.0 - z_eb, 1.0 - z_ha, 1.0 - z_hb],
                         axis=0).astype(bf)
    wz = jax.lax.dot_general(zn, ltb, _MM,
                             preferred_element_type=f32)

    rank_ea = _rank_from(z_ea, u_ea, wz[0 * NB:1 * NB])
    rank_eb = _rank_from(z_eb, u_eb, wz[1 * NB:2 * NB])
    rank_ha = _rank_from(z_ha, u_ha, wz[2 * NB:3 * NB])
    rank_hb = _rank_from(z_hb, u_hb, wz[3 * NB:4 * NB])

    g_mu = []
    g_var = []
    for i in range(NB):
        S = jnp.concatenate([
            _sel_rows(rank_ea[i:i + 1], K_EASY),
            _sel_rows(rank_eb[i:i + 1], K_EASY),
            _sel_rows(rank_ha[i:i + 1], K_HARD),
            _sel_rows(rank_hb[i:i + 1], K_HARD),
        ], axis=0)
        g_mu.append(jnp.dot(S, mu3[i]))
        g_var.append(jnp.dot(S, var3[i]))

    q_mu = jnp.concatenate([g[:2 * K_EASY] for g in g_mu], axis=0)
    p_mu = jnp.concatenate([g[2 * K_EASY:] for g in g_mu], axis=0)
    q_var = jnp.concatenate([g[:2 * K_EASY] for g in g_var], axis=0)
    p_var = jnp.concatenate([g[2 * K_EASY:] for g in g_var], axis=0)
    nq, np_ = 2 * K_EASY, 2 * K_HARD
    cq = q_var + 1e-5
    cp = p_var + 1e-5
    rq = 1.0 / cq
    slog_q = jnp.sum(jnp.log(cq), axis=-1)
    slog_p = jnp.sum(jnp.log(cp), axis=-1)
    q2r = jnp.sum(q_mu * q_mu * rq, axis=-1)
    m1 = jax.lax.dot_general(p_mu, q_mu * rq, _DN)
    m2 = jax.lax.dot_general(p_mu * p_mu, rq, _DN)
    m3 = jax.lax.dot_general(cp, rq, _DN)
    la = 0.0
    lb = 0.0
    for i in range(NB):
        rs, cs = i * np_, i * nq
        q2r_i = q2r[cs:cs + nq]
        t2 = slog_q[None, cs:cs + nq] - slog_p[rs:rs + np_, None]
        t1 = (q2r_i[None, :] - 2.0 * m1[rs:rs + np_, cs:cs + nq]
              + m2[rs:rs + np_, cs:cs + nq])
        dist = 0.5 * (t1 + t2 + m3[rs:rs + np_, cs:cs + nq]) - 0.5 * D
        val = 1.0 / (dist + 1.0)
        pos_a = jnp.mean(val[:K_HARD, :K_EASY])
        neg_a = jnp.mean(val[:K_HARD, K_EASY:])
        pos_b = jnp.mean(val[K_HARD:, K_EASY:])
        neg_b = jnp.mean(val[K_HARD:, :K_EASY])
        la += jnp.log(pos_a) + jnp.log(1.0 - neg_a)
        lb += jnp.log(pos_b) + jnp.log(1.0 - neg_b)

    ones_w = jnp.ones((2 * (D // 4), 128), bf)

    def _rowsum(x):
        p = (x[:, 0:128] + x[:, 128:256]) + (x[:, 256:384] + x[:, 384:512])
        hi = p.astype(bf)
        lo = (p - hi.astype(f32)).astype(bf)
        hl = jnp.concatenate([hi, lo], axis=1)
        return jax.lax.dot_general(hl, ones_w, _MM,
                                   preferred_element_type=f32)

    mu = mu3.reshape(NB * T, D)
    cf = cf3.reshape(NB * T, D)
    s1 = _rowsum(mu * cf)
    n1 = jnp.maximum(jnp.sqrt(_rowsum(mu * mu)), 1e-12)
    n2 = jnp.maximum(jnp.sqrt(_rowsum(cf * cf)), 1e-12)
    sim = (s1 / (n1 * n2) + 1.0) * 0.5
    sim_sum = jnp.sum(sim) * (1.0 / 128.0)
    return sim_sum, la, lb


def _loss_kernel(ltb_ref, act_ref, drop_ref, mu_ref, var_ref,
                 clip_ref, tf_ref, out_ref):
    b = pl.program_id(0)

    @pl.when(b == 0)
    def _init():
        tf = tf_ref[...]
        tn = jnp.maximum(
            jnp.sqrt(jnp.sum(tf * tf, axis=-1, keepdims=True)), 1e-12)
        e = tf / tn
        g = jax.lax.dot_general(e, e, _DN,
                                precision=jax.lax.Precision.HIGHEST)
        ii = jax.lax.broadcasted_iota(jnp.int32, (NCLS, NCLS), 0)
        jj = jax.lax.broadcasted_iota(jnp.int32, (NCLS, NCLS), 1)
        g = g - jnp.where(ii == jj, 1.0, 0.0)
        ortho = jnp.sqrt(jnp.sum(g * g))
        lane0 = jax.lax.broadcasted_iota(jnp.int32, (1, 8), 1)
        out_ref[...] = ortho * (lane0 == 3)

    sim_sum, la, lb = _step(act_ref[:, 0, :], drop_ref[:, 0, :], mu_ref[...],
                            var_ref[...], clip_ref[...], ltb_ref[...])

    lane = jax.lax.broadcasted_iota(jnp.int32, (1, 8), 1)
    contrib = (sim_sum * (lane == 0) + la * (lane == 1) + lb * (lane == 2))
    out_ref[...] = out_ref[...] + contrib


@jax.jit
def kernel(attn, mu_v, var_v, text_feat, mu_clip, labels, drop_mask):
    del labels
    act = attn.reshape(B, 1, T)
    drop3 = drop_mask.reshape(B, 1, T)
    ii = jax.lax.broadcasted_iota(jnp.int32, (T, T), 0)
    jj = jax.lax.broadcasted_iota(jnp.int32, (T, T), 1)
    ltf = (ii < jj).astype(jnp.float32)
    ltb = ltf.astype(jnp.bfloat16)
    acc = pl.pallas_call(
        _loss_kernel,
        grid=(B // NB,),
        in_specs=[
            pl.BlockSpec((T, T), lambda b: (0, 0)),
            pl.BlockSpec((NB, 1, T), lambda b: (b, 0, 0)),
            pl.BlockSpec((NB, 1, T), lambda b: (b, 0, 0)),
            pl.BlockSpec((NB, T, D), lambda b: (b, 0, 0)),
            pl.BlockSpec((NB, T, D), lambda b: (b, 0, 0)),
            pl.BlockSpec((NB, T, D), lambda b: (b, 0, 0)),
            pl.BlockSpec((NCLS, D), lambda b: (0, 0)),
        ],
        out_specs=pl.BlockSpec((1, 8), lambda b: (0, 0)),
        out_shape=jax.ShapeDtypeStruct((1, 8), jnp.float32),
    )(ltb, act, drop3, mu_v, var_v, mu_clip, text_feat)

    d_loss = A4 * -jnp.log(acc[0, 0] / (B * T))
    a_loss = A5 * (-acc[0, 1] / B)
    b_loss = A6 * (-acc[0, 2] / B)
    o_loss = A7 * acc[0, 3]
    total = d_loss + a_loss + b_loss + o_loss
    return total, d_loss, a_loss, b_loss, o_loss
